# baseline (speedup 1.0000x reference)
import functools
import math

import jax
import jax.numpy as jnp
from jax import lax
from jax.experimental import pallas as pl
from jax.experimental.pallas import tpu as pltpu
from jax.experimental.pallas import tpu_sc as plsc

N_ROWS = 1000000
C = 32
B = 4096
L = 200
LT = L // 8
HB = 64

_info = plsc.get_sparse_core_info()
NC = _info.num_cores
NS = _info.num_subcores
LANES = _info.num_lanes
NW = NC * NS
B_PER_W = B // NW

SCALE = math.sqrt(C)

_mesh = plsc.VectorSubcoreMesh(core_axis_name="c", subcore_axis_name="s")


@functools.partial(
    pl.kernel,
    mesh=_mesh,
    out_type=jax.ShapeDtypeStruct((C, LT, NW, 8, 128), jnp.float32),
    compiler_params=pltpu.CompilerParams(
        needs_layout_passes=False, use_tc_tiling_on_sc=True
    ),
    scratch_types=[
        pltpu.VMEM((LT, 8, B_PER_W), jnp.int32),
        pltpu.VMEM((HB, 128), jnp.float32),
        pltpu.VMEM((HB, 128), jnp.float32),
        pltpu.VMEM((2, C, 8, 128), jnp.float32),
        pltpu.VMEM((HB,), jnp.int32),
        pltpu.VMEM((HB,), jnp.int32),
        pltpu.SemaphoreType.DMA,
        pltpu.SemaphoreType.DMA,
        pltpu.SemaphoreType.DMA,
    ],
)
def _emb_kernel(
    xr_hbm, w_hbm, out_hbm,
    idx_v, rows0, rows1, outb, gb0, gb1,
    sem_g0, sem_g1, sem_o,
):
    wid = lax.axis_index("s") * NC + lax.axis_index("c")

    pltpu.sync_copy(xr_hbm.at[wid], idx_v)

    lanes_iota = lax.iota(jnp.int32, LANES)

    def fire_gather(lt, l8, bh, rows, sem, gbuf):
        for k in range(HB // LANES):
            v = idx_v[lt, l8, pl.ds(bh * HB + k * LANES, LANES)]
            gbuf[pl.ds(k * LANES, LANES)] = lax.shift_right_logical(v, 2)
        pltpu.async_copy(w_hbm.at[gbuf], rows, sem)

    def wait_gather(rows, sem):
        pltpu.make_async_copy(w_hbm.at[pl.ds(0, HB)], rows, sem).wait()

    def transpose(rows, buf, lt, l8, bh):
        buf_splat = jnp.full((LANES,), buf, jnp.int32)
        l_splat = jnp.full((LANES,), l8, jnp.int32)
        lt_splat = jnp.full((LANES,), lt, jnp.int32)

        def tbody(b0, carry):
            for dd in range(LANES):
                bvec = b0 + jnp.bitwise_and(lanes_iota + dd, LANES - 1)
                bgvec = bvec + bh * HB
                xv = plsc.load_gather(idx_v, [lt_splat, l_splat, bgvec])
                qoff = lax.shift_left(jnp.bitwise_and(xv, 3), 5)
                for c0 in range(0, C, LANES):
                    cvec = lanes_iota + c0
                    g = plsc.load_gather(rows, [bvec, qoff + cvec])
                    plsc.store_scatter(
                        outb, [buf_splat, cvec, l_splat, bgvec], g * SCALE
                    )
            return carry

        lax.fori_loop(0, HB // LANES, lambda i, c: tbody(i * LANES, c),
                      0, unroll=2)

    def fire_out(buf, lt):
        pltpu.async_copy(outb.at[buf], out_hbm.at[:, lt, wid], sem_o)

    def wait_out():
        pltpu.make_async_copy(outb.at[0], out_hbm.at[:, 0, wid], sem_o).wait()

    fire_gather(0, 0, 0, rows0, sem_g0, gb0)
    fire_gather(0, 0, 1, rows1, sem_g1, gb1)

    def body(k, carry):
        lt = lax.shift_right_logical(k, 3)
        l8 = jnp.bitwise_and(k, 7)
        buf = jnp.bitwise_and(lt, 1)
        nk = jnp.minimum(k + 1, LT * 8 - 1)
        nlt = lax.shift_right_logical(nk, 3)
        nl8 = jnp.bitwise_and(nk, 7)
        for bh in range(2):
            rows = rows0 if bh == 0 else rows1
            sem = sem_g0 if bh == 0 else sem_g1
            gbuf = gb0 if bh == 0 else gb1
            wait_gather(rows, sem)
            transpose(rows, buf, lt, l8, bh)
            fire_gather(nlt, nl8, bh, rows, sem, gbuf)

        @pl.when(l8 == 7)
        def _():
            @pl.when(lt > 0)
            def _():
                wait_out()

            fire_out(buf, lt)

        return carry

    lax.fori_loop(0, LT * 8, body, 0)

    wait_out()
    wait_gather(rows0, sem_g0)
    wait_gather(rows1, sem_g1)


def kernel(x, emb_weight):
    xr = (
        x.astype(jnp.int32)
        .reshape(NW, B_PER_W, LT, 8)
        .transpose(0, 2, 3, 1)
    )
    out5 = _emb_kernel(xr, emb_weight.reshape(N_ROWS // 4, 128))
    return out5.transpose(2, 4, 0, 1, 3).reshape(B, C, L)


# speedup vs baseline: 1.2288x; 1.2288x over previous
import functools
import math

import jax
import jax.numpy as jnp
from jax import lax
from jax.experimental import pallas as pl
from jax.experimental.pallas import tpu as pltpu
from jax.experimental.pallas import tpu_sc as plsc

N_ROWS = 1000000
C = 32
B = 4096
L = 200
LT = L // 8

_info = plsc.get_sparse_core_info()
NC = _info.num_cores
NS = _info.num_subcores
LANES = _info.num_lanes
NW = NC * NS
B_PER_W = B // NW

SCALE = math.sqrt(C)

_mesh = plsc.VectorSubcoreMesh(core_axis_name="c", subcore_axis_name="s")


@functools.partial(
    pl.kernel,
    mesh=_mesh,
    out_type=jax.ShapeDtypeStruct((C, LT, NW, 8, 128), jnp.float32),
    compiler_params=pltpu.CompilerParams(
        needs_layout_passes=False, use_tc_tiling_on_sc=False
    ),
    scratch_types=[
        pltpu.VMEM((LT, 8, B_PER_W), jnp.int32),
        pltpu.VMEM((B_PER_W, C), jnp.float32),
        pltpu.VMEM((B_PER_W, C), jnp.float32),
        pltpu.VMEM((2, C, 8, 128), jnp.float32),
        pltpu.SemaphoreType.DMA,
        pltpu.SemaphoreType.DMA,
        pltpu.SemaphoreType.DMA,
    ],
)
def _emb_kernel(
    xr_hbm, w_hbm, out_hbm,
    idx_v, rows0, rows1, outb,
    sem_g0, sem_g1, sem_o,
):
    wid = lax.axis_index("s") * NC + lax.axis_index("c")

    pltpu.sync_copy(xr_hbm.at[wid], idx_v)

    lanes_iota = lax.iota(jnp.int32, LANES)

    def fire_gather(lt, l8, rows, sem):
        pltpu.async_copy(w_hbm.at[idx_v.at[lt, l8]], rows, sem)

    def wait_gather(rows, sem):
        pltpu.make_async_copy(w_hbm.at[pl.ds(0, B_PER_W)], rows, sem).wait()

    def transpose(rows, buf, l8):
        buf_splat = jnp.full((LANES,), buf, jnp.int32)
        l_splat = jnp.full((LANES,), l8, jnp.int32)

        def tbody(b0, carry):
            for c0 in range(0, C, LANES):
                cvec = lanes_iota + c0
                for dd in range(LANES):
                    bvec = b0 + jnp.bitwise_and(lanes_iota + dd, LANES - 1)
                    g = plsc.load_gather(rows, [bvec, cvec])
                    plsc.store_scatter(
                        outb, [buf_splat, cvec, l_splat, bvec], g * SCALE
                    )
            return carry

        lax.fori_loop(0, B_PER_W // LANES, lambda i, c: tbody(i * LANES, c),
                      0, unroll=2)

    def fire_out(buf, lt):
        pltpu.async_copy(outb.at[buf], out_hbm.at[:, lt, wid], sem_o)

    def wait_out():
        pltpu.make_async_copy(outb.at[0], out_hbm.at[:, 0, wid], sem_o).wait()

    fire_gather(0, 0, rows0, sem_g0)
    fire_gather(0, 1, rows1, sem_g1)

    def body(lt, carry):
        buf = lax.rem(lt, 2)
        nlt = jnp.minimum(lt + 1, LT - 1)
        for l8 in range(8):
            rows = rows0 if l8 % 2 == 0 else rows1
            sem = sem_g0 if l8 % 2 == 0 else sem_g1
            wait_gather(rows, sem)
            transpose(rows, buf, l8)
            if l8 < 6:
                fire_gather(lt, l8 + 2, rows, sem)
            else:
                fire_gather(nlt, l8 - 6, rows, sem)

        @pl.when(lt > 0)
        def _():
            wait_out()

        fire_out(buf, lt)
        return carry

    lax.fori_loop(0, LT, body, 0)

    wait_out()
    wait_gather(rows0, sem_g0)
    wait_gather(rows1, sem_g1)


def kernel(x, emb_weight):
    xr = (
        x.astype(jnp.int32)
        .reshape(NW, B_PER_W, LT, 8)
        .transpose(0, 2, 3, 1)
    )
    out5 = _emb_kernel(xr, emb_weight)
    return out5.transpose(2, 4, 0, 1, 3).reshape(B, C, L)
